# baseline (speedup 1.0000x reference)
import functools

import jax
import jax.numpy as jnp
from jax import lax
from jax.experimental import pallas as pl
from jax.experimental.pallas import tpu as pltpu
from jax.experimental.pallas import tpu_sc as plsc

D = 64
N = 819200
NC, NS, L = 2, 16, 16
NW = NC * NS
PER_W = N // NW
CH = 512
NCHUNK = PER_W // CH

_mesh = plsc.VectorSubcoreMesh(
    core_axis_name="c", subcore_axis_name="s", num_cores=NC, num_subcores=NS
)


@functools.partial(
    pl.kernel,
    out_type=jax.ShapeDtypeStruct((N, D), jnp.float32),
    mesh=_mesh,
    scratch_types=[
        pltpu.VMEM((CH,), jnp.int32),
        pltpu.VMEM((CH,), jnp.int32),
        pltpu.VMEM((CH,), jnp.float32),
        pltpu.VMEM((500, D), jnp.float32),
        pltpu.VMEM((CH, D), jnp.float32),
        pltpu.VMEM((D,), jnp.float32),
    ],
    compiler_params=pltpu.CompilerParams(use_tc_tiling_on_sc=False),
)
def _encode(types_h, nodes_h, vals_h, table_h, w_h, out_h,
            types_v, nodes_v, vals_v, table_v, rows_v, w_v):
    cid = lax.axis_index("c")
    sid = lax.axis_index("s")
    wid = sid * NC + cid
    base = wid * PER_W
    pltpu.sync_copy(w_h, w_v)
    pltpu.sync_copy(table_h, table_v)

    def chunk(ci, carry):
        off = base + ci * CH
        pltpu.sync_copy(types_h.at[pl.ds(off, CH)], types_v)
        pltpu.sync_copy(nodes_h.at[pl.ds(off, CH)], nodes_v)
        pltpu.sync_copy(vals_h.at[pl.ds(off, CH)], vals_v)

        wjs = [w_v[pl.ds(j * L, L)] for j in range(D // L)]

        def grpfn(g16, _):
            i0 = g16 * L
            fused = types_v[pl.ds(i0, L)] * 100 + nodes_v[pl.ds(i0, L)]
            vv = vals_v[pl.ds(i0, L)]
            for k in range(L):
                r = fused[k]
                v = vv[k]
                for j in range(D // L):
                    sl = pl.ds(j * L, L)
                    rows_v[i0 + k, sl] = table_v[r, sl] + v * wjs[j]
            return 0
        lax.fori_loop(0, CH // L, grpfn, 0)

        pltpu.sync_copy(rows_v, out_h.at[pl.ds(off, CH)])
        return carry

    lax.fori_loop(0, NCHUNK, chunk, 0)


def kernel(token_types, token_values, node_indices, token_table, node_table,
           value_W, value_b):
    table = (token_table[:, None, :] + node_table[None, :, :]
             + value_b[None, None, :]).reshape(500, D)
    vals = token_values[:, 0]
    w = value_W[:, 0]
    return _encode(token_types.astype(jnp.int32), node_indices.astype(jnp.int32),
                   vals, table, w)


# speedup vs baseline: 1.3095x; 1.3095x over previous
import functools

import jax
import jax.numpy as jnp
from jax import lax
from jax.experimental import pallas as pl
from jax.experimental.pallas import tpu as pltpu
from jax.experimental.pallas import tpu_sc as plsc

D = 64
N = 819200
NC, NS, L = 2, 16, 16
NW = NC * NS
PER_W = N // NW
CH = 512
NG = CH // 128
NCHUNK = PER_W // CH

_mesh = plsc.VectorSubcoreMesh(
    core_axis_name="c", subcore_axis_name="s", num_cores=NC, num_subcores=NS
)


@functools.partial(
    pl.kernel,
    out_type=jax.ShapeDtypeStruct((N, D), jnp.float32),
    mesh=_mesh,
    scratch_types=[
        pltpu.VMEM((2, CH), jnp.int32),
        pltpu.VMEM((2, CH), jnp.int32),
        pltpu.VMEM((2, CH), jnp.float32),
        pltpu.VMEM((2, NG, 128), jnp.int32),
        pltpu.VMEM((2, CH, D), jnp.float32),
        pltpu.VMEM((D,), jnp.float32),
        pltpu.SemaphoreType.DMA,
        pltpu.SemaphoreType.DMA,
        pltpu.SemaphoreType.DMA,
        pltpu.SemaphoreType.DMA,
        pltpu.SemaphoreType.DMA,
        pltpu.SemaphoreType.DMA,
    ],
    compiler_params=pltpu.CompilerParams(use_tc_tiling_on_sc=False),
)
def _encode(types_h, nodes_h, vals_h, table_h, w_h, out_h,
            types_v, nodes_v, vals_v, idx_v, rows_v, w_v,
            sin0, sin1, sgat0, sgat1, sout0, sout1):
    sins = (sin0, sin1)
    sgats = (sgat0, sgat1)
    souts = (sout0, sout1)
    cid = lax.axis_index("c")
    sid = lax.axis_index("s")
    base = (sid * NC + cid) * PER_W

    pltpu.sync_copy(w_h, w_v)

    def in_copies(ci, b):
        off = base + ci * CH
        return (
            pltpu.make_async_copy(types_h.at[pl.ds(off, CH)], types_v.at[b], sins[b]),
            pltpu.make_async_copy(nodes_h.at[pl.ds(off, CH)], nodes_v.at[b], sins[b]),
            pltpu.make_async_copy(vals_h.at[pl.ds(off, CH)], vals_v.at[b], sins[b]),
        )

    def idx_compute(b):
        for g in range(NG):
            def fuse(k, _, g=g):
                sl = pl.ds(g * 128 + k * L, L)
                idx_v[b, g, pl.ds(k * L, L)] = (
                    types_v[b, sl] * 100 + nodes_v[b, sl]
                )
                return 0
            lax.fori_loop(0, 128 // L, fuse, 0)

    def gather_copies(b):
        return [
            pltpu.make_async_copy(
                table_h.at[idx_v.at[b, g]],
                rows_v.at[b, pl.ds(g * 128, 128)],
                sgats[b],
            )
            for g in range(NG)
        ]

    def out_copy(ci, b):
        off = base + ci * CH
        return pltpu.make_async_copy(rows_v.at[b], out_h.at[pl.ds(off, CH)], souts[b])

    def fma(b):
        def grpfn(g16, _):
            i0 = g16 * L
            vv = vals_v[b, pl.ds(i0, L)]
            wjs = [w_v[pl.ds(j * L, L)] for j in range(D // L)]
            for k in range(L):
                v = vv[k]
                for j in range(D // L):
                    plsc.addupdate(rows_v.at[b, i0 + k, pl.ds(j * L, L)], v * wjs[j])
            return 0
        lax.fori_loop(0, CH // L, grpfn, 0)

    for dsc in in_copies(0, 0):
        dsc.start()
    for dsc in in_copies(1, 1):
        dsc.start()
    for dsc in in_copies(0, 0):
        dsc.wait()
    idx_compute(0)
    for dsc in gather_copies(0):
        dsc.start()

    def pair(p, carry):
        for b in (0, 1):
            ci = 2 * p + b
            nb = 1 - b

            @pl.when(ci + 1 < NCHUNK)
            def _():
                for dsc in in_copies(ci + 1, nb):
                    dsc.wait()
                idx_compute(nb)

                @pl.when(ci >= 1)
                def _():
                    out_copy(ci - 1, nb).wait()

                for dsc in gather_copies(nb):
                    dsc.start()

            for dsc in gather_copies(b):
                dsc.wait()
            fma(b)
            out_copy(ci, b).start()

            @pl.when(ci + 2 < NCHUNK)
            def _():
                for dsc in in_copies(ci + 2, b):
                    dsc.start()
        return carry

    lax.fori_loop(0, NCHUNK // 2, pair, 0)

    out_copy(NCHUNK - 2, 0).wait()
    out_copy(NCHUNK - 1, 1).wait()


def kernel(token_types, token_values, node_indices, token_table, node_table,
           value_W, value_b):
    table = (token_table[:, None, :] + node_table[None, :, :]
             + value_b[None, None, :]).reshape(500, D)
    vals = token_values[:, 0]
    w = value_W[:, 0]
    return _encode(token_types.astype(jnp.int32), node_indices.astype(jnp.int32),
                   vals, table, w)


# speedup vs baseline: 1.6535x; 1.2627x over previous
import functools

import jax
import jax.numpy as jnp
from jax import lax
from jax.experimental import pallas as pl
from jax.experimental.pallas import tpu as pltpu
from jax.experimental.pallas import tpu_sc as plsc

D = 64
N = 819200
NC, NS, L = 2, 16, 16
NW = NC * NS
PER_W = N // NW
CH = 512
TS = 80
NCHUNK = PER_W // CH

_mesh = plsc.VectorSubcoreMesh(
    core_axis_name="c", subcore_axis_name="s", num_cores=NC, num_subcores=NS
)


@functools.partial(
    pl.kernel,
    out_type=jax.ShapeDtypeStruct((D, N), jnp.float32),
    mesh=_mesh,
    scratch_types=[
        pltpu.VMEM((2, CH), jnp.int32),
        pltpu.VMEM((2, CH), jnp.int32),
        pltpu.VMEM((2, CH), jnp.float32),
        pltpu.VMEM((500 * TS,), jnp.float32),
        pltpu.VMEM((2, D, CH), jnp.float32),
        pltpu.VMEM((D,), jnp.float32),
        pltpu.VMEM((D * L,), jnp.float32),
        pltpu.SemaphoreType.DMA,
        pltpu.SemaphoreType.DMA,
        pltpu.SemaphoreType.DMA,
        pltpu.SemaphoreType.DMA,
    ],
    compiler_params=pltpu.CompilerParams(
        use_tc_tiling_on_sc=True, needs_layout_passes=False
    ),
)
def _encode(types_h, nodes_h, vals_h, table_h, w_h, out_h,
            types_v, nodes_v, vals_v, table_v, stage_v, w_v, wrot_v,
            sin0, sin1, sout0, sout1):
    sins = (sin0, sin1)
    souts = (sout0, sout1)
    cid = lax.axis_index("c")
    sid = lax.axis_index("s")
    base = (sid * NC + cid) * PER_W

    pltpu.sync_copy(w_h, w_v)
    pltpu.sync_copy(table_h, table_v)

    iota = lax.iota(jnp.int32, L)

    def pre(j, _):
        col = (iota + j) & 63
        wrot_v[pl.ds(j * L, L)] = plsc.load_gather(w_v, [col])
        return 0
    lax.fori_loop(0, D, pre, 0)

    def in_copies(ci, b):
        off = base + ci * CH
        return (
            pltpu.make_async_copy(types_h.at[pl.ds(off, CH)],
                                  types_v.at[b], sins[b]),
            pltpu.make_async_copy(nodes_h.at[pl.ds(off, CH)],
                                  nodes_v.at[b], sins[b]),
            pltpu.make_async_copy(vals_h.at[pl.ds(off, CH)],
                                  vals_v.at[b], sins[b]),
        )

    def out_copy(ci, b):
        off = base + ci * CH
        return pltpu.make_async_copy(
            stage_v.at[b], out_h.at[:, pl.ds(off, CH)], souts[b]
        )

    def compute(b):
        def grpfn(g16, _):
            t0 = g16 * L
            sl = pl.ds(t0, L)
            fgl = (types_v[b, sl] * 100 + nodes_v[b, sl]) * TS + iota
            vv = vals_v[b, sl]
            tokv = iota + t0
            for j in range(D):
                row = plsc.load_gather(table_v, [fgl + j])
                wr = wrot_v[pl.ds(j * L, L)]
                col = (iota + j) & 63
                plsc.store_scatter(stage_v.at[b], [col, tokv], row + vv * wr)
            return 0
        lax.fori_loop(0, CH // L, grpfn, 0)

    for dsc in in_copies(0, 0):
        dsc.start()
    for dsc in in_copies(1, 1):
        dsc.start()

    def pair(p, carry):
        for b in (0, 1):
            ci = 2 * p + b
            for dsc in in_copies(ci, b):
                dsc.wait()

            @pl.when(ci >= 2)
            def _():
                out_copy(ci - 2, b).wait()

            compute(b)
            out_copy(ci, b).start()

            @pl.when(ci + 2 < NCHUNK)
            def _():
                for dsc in in_copies(ci + 2, b):
                    dsc.start()
        return carry

    lax.fori_loop(0, NCHUNK // 2, pair, 0)

    out_copy(NCHUNK - 2, 0).wait()
    out_copy(NCHUNK - 1, 1).wait()


def kernel(token_types, token_values, node_indices, token_table, node_table,
           value_W, value_b):
    table = (token_table[:, None, :] + node_table[None, :, :]
             + value_b[None, None, :]).reshape(500, D)
    table80 = jnp.concatenate([table, table[:, : TS - D]], axis=1)
    vals = token_values[:, 0]
    w = value_W[:, 0]
    out_t = _encode(token_types.astype(jnp.int32), node_indices.astype(jnp.int32),
                    vals, table80.reshape(500 * TS), w)
    return out_t.T
